# baseline (speedup 1.0000x reference)
import functools

import jax
import jax.numpy as jnp
from jax import lax
from jax.experimental import pallas as pl
from jax.experimental.pallas import tpu as pltpu
from jax.experimental.pallas import tpu_sc as plsc

D_MODEL = 224
NV = 1955
NSEG = 2
EPS = 1e-5
LANES = 16
NUM_CORES = 2
NUM_SUBCORES = 16
NW = NUM_CORES * NUM_SUBCORES
NJ = D_MODEL // LANES
UNROLL = 2


def _comb_body(word_ref, seg_ref, out_ref):
    out_ref[0] = word_ref[...] + seg_ref[0:1, :]
    out_ref[1] = word_ref[...] + seg_ref[1:2, :]


def _build_comb(word_emb, seg_emb):
    comb = pl.pallas_call(
        _comb_body,
        out_shape=jax.ShapeDtypeStruct((NSEG, NV, D_MODEL), jnp.float32),
    )(word_emb, seg_emb)
    return comb.reshape(NSEG * NV, D_MODEL)


_GDN = lax.GatherDimensionNumbers(
    offset_dims=(), collapsed_slice_dims=(0,), start_index_map=(0,))


def _shuffle(v, idx):
    return lax.gather(v, idx[:, None], _GDN, slice_sizes=(1,),
                      mode=lax.GatherScatterMode.PROMISE_IN_BOUNDS)


def _xlane_sum(v):
    idx = lax.iota(jnp.int32, LANES)
    for sh in (1, 2, 4, 8):
        v = v + _shuffle(v, idx ^ sh)
    return v


def _rsqrt(v):
    i = lax.bitcast_convert_type(v, jnp.int32)
    y = lax.bitcast_convert_type(0x5F3759DF - (i >> 1), jnp.float32)
    for _ in range(2):
        y = y * (1.5 - 0.5 * v * y * y)
    return y


def _ln_tokens(rows, i0, gv, bv):
    hs, avec, cvec = [], [], []
    for u in range(UNROLL):
        i = i0 + u
        h = [rows[i, pl.ds(j * LANES, LANES)] for j in range(NJ)]
        svec = h[0]
        qvec = h[0] * h[0]
        for j in range(1, NJ):
            svec = svec + h[j]
            qvec = qvec + h[j] * h[j]
        sb = jnp.full((LANES,), jnp.sum(svec))
        qb = jnp.full((LANES,), jnp.sum(qvec))
        meanv = sb * (1.0 / D_MODEL)
        varv = qb * (1.0 / D_MODEL) - meanv * meanv + EPS
        rstd = _rsqrt(varv)
        hs.append(h)
        avec.append(rstd)
        cvec.append(meanv)
    for j in range(NJ):
        sl = pl.ds(j * LANES, LANES)
        gj = gv[sl]
        bj = bv[sl]
        for u in range(UNROLL):
            rows[i0 + u, sl] = (hs[u][j] - cvec[u]) * (avec[u] * gj) + bj


def _make_sc_kernel(n_tok, k_chunk):
    per_w = n_tok // NW
    n_chunks = per_w // k_chunk
    mesh = plsc.VectorSubcoreMesh(
        core_axis_name="c", subcore_axis_name="s",
        num_cores=NUM_CORES, num_subcores=NUM_SUBCORES,
    )

    @functools.partial(
        pl.kernel,
        mesh=mesh,
        compiler_params=pltpu.CompilerParams(
            use_tc_tiling_on_sc=False, needs_layout_passes=False),
        out_type=jax.ShapeDtypeStruct((n_tok, D_MODEL), jnp.float32),
        scratch_types=[
            pltpu.VMEM((per_w,), jnp.int32),
            pltpu.VMEM((per_w,), jnp.int32),
            pltpu.VMEM((n_chunks, k_chunk), jnp.int32),
            pltpu.VMEM((k_chunk, D_MODEL), jnp.float32),
            pltpu.VMEM((k_chunk, D_MODEL), jnp.float32),
            pltpu.VMEM((k_chunk, D_MODEL), jnp.float32),
            pltpu.VMEM((D_MODEL,), jnp.float32),
            pltpu.VMEM((D_MODEL,), jnp.float32),
            pltpu.SemaphoreType.DMA,
            pltpu.SemaphoreType.DMA,
            pltpu.SemaphoreType.DMA,
            pltpu.SemaphoreType.DMA,
            pltpu.SemaphoreType.DMA,
            pltpu.SemaphoreType.DMA,
        ],
    )
    def sc_kernel(comb_hbm, xf_hbm, segf_hbm, gamma_hbm, beta_hbm, out_hbm,
                  xall, sall, cvall, rows0, rows1, rows2, gv, bv,
                  gsem0, gsem1, gsem2, osem0, osem1, osem2):
        wid = lax.axis_index("s") * NUM_CORES + lax.axis_index("c")
        base = wid * per_w
        pltpu.sync_copy(gamma_hbm, gv)
        pltpu.sync_copy(beta_hbm, bv)
        pltpu.sync_copy(xf_hbm.at[pl.ds(base, per_w)], xall)
        pltpu.sync_copy(segf_hbm.at[pl.ds(base, per_w)], sall)

        def cvbody(c, carry):
            for k in range(k_chunk // LANES):
                fl = pl.ds(c * k_chunk + k * LANES, LANES)
                cvall[c, pl.ds(k * LANES, LANES)] = sall[fl] * NV + xall[fl]
            return carry

        lax.fori_loop(0, n_chunks, cvbody, 0)

        rows = (rows0, rows1, rows2)
        gsem = (gsem0, gsem1, gsem2)
        osem = (osem0, osem1, osem2)

        def fire_gather(c, b):
            pltpu.async_copy(comb_hbm.at[cvall.at[c]], rows[b], gsem[b])

        def wait_gather(c, b):
            pltpu.make_async_copy(comb_hbm.at[cvall.at[c]], rows[b],
                                  gsem[b]).wait()

        def fire_scatter(c, b):
            pltpu.async_copy(rows[b], out_hbm.at[pl.ds(base + c * k_chunk,
                                                       k_chunk)], osem[b])

        def wait_scatter(c, b):
            pltpu.make_async_copy(rows[b],
                                  out_hbm.at[pl.ds(base + c * k_chunk,
                                                   k_chunk)], osem[b]).wait()

        def process(c, b):
            wait_gather(c, b)

            def tok(i, tcarry):
                _ln_tokens(rows[b], i * UNROLL, gv, bv)
                return tcarry

            lax.fori_loop(0, k_chunk // UNROLL, tok, 0)
            fire_scatter(c, b)

        fire_gather(0, 0)
        fire_gather(1, 1)
        process(0, 0)
        fire_gather(2, 2)
        process(1, 1)
        wait_scatter(0, 0)
        fire_gather(3, 0)

        def round_body(r, carry):
            for b in range(3):
                c = r * 3 + b + 2
                process(c, (b + 2) % 3)
                wait_scatter(c - 1, (b + 1) % 3)
                fire_gather(c + 2, (b + 1) % 3)
            return carry

        lax.fori_loop(0, (n_chunks - 4) // 3, round_body, 0)
        for c in range(n_chunks - 2 - ((n_chunks - 4) % 3), n_chunks):
            if c + 2 < n_chunks:
                process(c, c % 3)
                wait_scatter(c - 1, (c - 1) % 3)
                fire_gather(c + 2, (c + 2) % 3)
            else:
                process(c, c % 3)
                wait_scatter(c - 1, (c - 1) % 3)
        wait_scatter(n_chunks - 1, (n_chunks - 1) % 3)

    return sc_kernel


def kernel(x, seg, word_emb, seg_emb, gamma, beta):
    b, l = x.shape
    n_tok = b * l
    comb = _build_comb(word_emb, seg_emb)
    xf = x.reshape(n_tok)
    segf = seg.reshape(n_tok)
    sc = _make_sc_kernel(n_tok, 128)
    out = sc(comb, xf, segf, gamma, beta)
    return out.reshape(b, l, D_MODEL)


# speedup vs baseline: 1.3487x; 1.3487x over previous
import functools

import jax
import jax.numpy as jnp
from jax import lax
from jax.experimental import pallas as pl
from jax.experimental.pallas import tpu as pltpu
from jax.experimental.pallas import tpu_sc as plsc

D_MODEL = 224
NV = 1955
NSEG = 2
EPS = 1e-5
LANES = 16
NUM_CORES = 2
NUM_SUBCORES = 16
NW = NUM_CORES * NUM_SUBCORES
NJ = D_MODEL // LANES
UNROLL = 8


def _comb_body(word_ref, seg_ref, out_ref):
    out_ref[0] = word_ref[...] + seg_ref[0:1, :]
    out_ref[1] = word_ref[...] + seg_ref[1:2, :]


def _build_comb(word_emb, seg_emb):
    comb = pl.pallas_call(
        _comb_body,
        out_shape=jax.ShapeDtypeStruct((NSEG, NV, D_MODEL), jnp.float32),
    )(word_emb, seg_emb)
    return comb.reshape(NSEG * NV, D_MODEL)


_GDN = lax.GatherDimensionNumbers(
    offset_dims=(), collapsed_slice_dims=(0,), start_index_map=(0,))


def _shuffle(v, idx):
    return lax.gather(v, idx[:, None], _GDN, slice_sizes=(1,),
                      mode=lax.GatherScatterMode.PROMISE_IN_BOUNDS)


def _xlane_sum(v):
    idx = lax.iota(jnp.int32, LANES)
    for sh in (1, 2, 4, 8):
        v = v + _shuffle(v, idx ^ sh)
    return v


def _rsqrt(v):
    i = lax.bitcast_convert_type(v, jnp.int32)
    y = lax.bitcast_convert_type(0x5F3759DF - (i >> 1), jnp.float32)
    for _ in range(2):
        y = y * (1.5 - 0.5 * v * y * y)
    return y


def _ln_tokens(rows, i0, gv, bv):
    hs, avec, cvec = [], [], []
    for u in range(UNROLL):
        i = i0 + u
        h = [rows[i, pl.ds(j * LANES, LANES)] for j in range(NJ)]
        svec = h[0]
        qvec = h[0] * h[0]
        for j in range(1, NJ):
            svec = svec + h[j]
            qvec = qvec + h[j] * h[j]
        sb = jnp.full((LANES,), jnp.sum(svec))
        qb = jnp.full((LANES,), jnp.sum(qvec))
        meanv = sb * (1.0 / D_MODEL)
        varv = qb * (1.0 / D_MODEL) - meanv * meanv + EPS
        rstd = _rsqrt(varv)
        hs.append(h)
        avec.append(rstd)
        cvec.append(meanv)
    for j in range(NJ):
        sl = pl.ds(j * LANES, LANES)
        gj = gv[sl]
        bj = bv[sl]
        for u in range(UNROLL):
            rows[i0 + u, sl] = (hs[u][j] - cvec[u]) * (avec[u] * gj) + bj


def _make_sc_kernel(n_tok, k_chunk):
    per_w = n_tok // NW
    n_chunks = per_w // k_chunk
    mesh = plsc.VectorSubcoreMesh(
        core_axis_name="c", subcore_axis_name="s",
        num_cores=NUM_CORES, num_subcores=NUM_SUBCORES,
    )

    @functools.partial(
        pl.kernel,
        mesh=mesh,
        compiler_params=pltpu.CompilerParams(
            use_tc_tiling_on_sc=False, needs_layout_passes=False),
        out_type=jax.ShapeDtypeStruct((n_tok, D_MODEL), jnp.float32),
        scratch_types=[
            pltpu.VMEM((per_w,), jnp.int32),
            pltpu.VMEM((per_w,), jnp.int32),
            pltpu.VMEM((n_chunks, k_chunk), jnp.int32),
            pltpu.VMEM((k_chunk, D_MODEL), jnp.float32),
            pltpu.VMEM((k_chunk, D_MODEL), jnp.float32),
            pltpu.VMEM((k_chunk, D_MODEL), jnp.float32),
            pltpu.VMEM((D_MODEL,), jnp.float32),
            pltpu.VMEM((D_MODEL,), jnp.float32),
            pltpu.SemaphoreType.DMA,
            pltpu.SemaphoreType.DMA,
            pltpu.SemaphoreType.DMA,
            pltpu.SemaphoreType.DMA,
            pltpu.SemaphoreType.DMA,
            pltpu.SemaphoreType.DMA,
        ],
    )
    def sc_kernel(comb_hbm, xf_hbm, segf_hbm, gamma_hbm, beta_hbm, out_hbm,
                  xall, sall, cvall, rows0, rows1, rows2, gv, bv,
                  gsem0, gsem1, gsem2, osem0, osem1, osem2):
        wid = lax.axis_index("s") * NUM_CORES + lax.axis_index("c")
        base = wid * per_w
        pltpu.sync_copy(gamma_hbm, gv)
        pltpu.sync_copy(beta_hbm, bv)
        pltpu.sync_copy(xf_hbm.at[pl.ds(base, per_w)], xall)
        pltpu.sync_copy(segf_hbm.at[pl.ds(base, per_w)], sall)

        def cvbody(c, carry):
            for k in range(k_chunk // LANES):
                fl = pl.ds(c * k_chunk + k * LANES, LANES)
                cvall[c, pl.ds(k * LANES, LANES)] = sall[fl] * NV + xall[fl]
            return carry

        lax.fori_loop(0, n_chunks, cvbody, 0)

        rows = (rows0, rows1, rows2)
        gsem = (gsem0, gsem1, gsem2)
        osem = (osem0, osem1, osem2)

        def fire_gather(c, b):
            pltpu.async_copy(comb_hbm.at[cvall.at[c]], rows[b], gsem[b])

        def wait_gather(c, b):
            pltpu.make_async_copy(comb_hbm.at[cvall.at[c]], rows[b],
                                  gsem[b]).wait()

        def fire_scatter(c, b):
            pltpu.async_copy(rows[b], out_hbm.at[pl.ds(base + c * k_chunk,
                                                       k_chunk)], osem[b])

        def wait_scatter(c, b):
            pltpu.make_async_copy(rows[b],
                                  out_hbm.at[pl.ds(base + c * k_chunk,
                                                   k_chunk)], osem[b]).wait()

        def process(c, b):
            wait_gather(c, b)

            def tok(i, tcarry):
                _ln_tokens(rows[b], i * UNROLL, gv, bv)
                return tcarry

            lax.fori_loop(0, k_chunk // UNROLL, tok, 0)
            fire_scatter(c, b)

        fire_gather(0, 0)
        fire_gather(1, 1)
        process(0, 0)
        fire_gather(2, 2)
        process(1, 1)
        wait_scatter(0, 0)
        fire_gather(3, 0)

        def round_body(r, carry):
            for b in range(3):
                c = r * 3 + b + 2
                process(c, (b + 2) % 3)
                wait_scatter(c - 1, (b + 1) % 3)
                fire_gather(c + 2, (b + 1) % 3)
            return carry

        lax.fori_loop(0, (n_chunks - 4) // 3, round_body, 0)
        for c in range(n_chunks - 2 - ((n_chunks - 4) % 3), n_chunks):
            if c + 2 < n_chunks:
                process(c, c % 3)
                wait_scatter(c - 1, (c - 1) % 3)
                fire_gather(c + 2, (c + 2) % 3)
            else:
                process(c, c % 3)
                wait_scatter(c - 1, (c - 1) % 3)
        wait_scatter(n_chunks - 1, (n_chunks - 1) % 3)

    return sc_kernel


def kernel(x, seg, word_emb, seg_emb, gamma, beta):
    b, l = x.shape
    n_tok = b * l
    comb = _build_comb(word_emb, seg_emb)
    xf = x.reshape(n_tok)
    segf = seg.reshape(n_tok)
    sc = _make_sc_kernel(n_tok, 128)
    out = sc(comb, xf, segf, gamma, beta)
    return out.reshape(b, l, D_MODEL)
